# baseline (speedup 1.0000x reference)
import numpy as np
import jax
import jax.numpy as jnp
from jax.experimental import pallas as pl
from jax.experimental.pallas import tpu as pltpu

_MAX_DETECTION_POINTS = 5000
_MAX_DETECTIONS_PER_IMAGE = 100
_NUM_CLASSES = 90
_IOU_THRESHOLD = 0.5
_IMAGE_SIZE = 512
_MIN_LEVEL = 3
_MAX_LEVEL = 7
_NUM_SCALES = 3
_ANCHOR_SCALE = 4.0
_ASPECTS = [(1.0, 1.0), (1.4, 0.7), (0.7, 1.4)]

_NP = 5120


def _anchor_table():
    all_boxes = []
    for level in range(_MIN_LEVEL, _MAX_LEVEL + 1):
        stride = 2 ** level
        feat = _IMAGE_SIZE // stride
        configs = []
        for octave in range(_NUM_SCALES):
            for (ax, ay) in _ASPECTS:
                base = _ANCHOR_SCALE * stride * (2.0 ** (octave / float(_NUM_SCALES)))
                configs.append((base * ax / 2.0, base * ay / 2.0))
        xs = (np.arange(feat, dtype=np.float32) + 0.5) * stride
        ys = (np.arange(feat, dtype=np.float32) + 0.5) * stride
        yv, xv = np.meshgrid(ys, xs, indexing='ij')
        level_boxes = np.zeros((feat, feat, len(configs), 4), dtype=np.float32)
        for a, (hx, hy) in enumerate(configs):
            level_boxes[:, :, a, 0] = yv - hy
            level_boxes[:, :, a, 1] = xv - hx
            level_boxes[:, :, a, 2] = yv + hy
            level_boxes[:, :, a, 3] = xv + hx
        all_boxes.append(level_boxes.reshape(-1, 4))
    return np.concatenate(all_boxes, axis=0)


_ANCHORS = jnp.asarray(_anchor_table())


def _nms_kernel(s_ref, box_ref, anc_ref, cl_ref, sc_ref, out_ref):
    logits = s_ref[0]
    cls_f = cl_ref[0]
    scale = sc_ref[0, 0, 0]

    ty = box_ref[0, 0]
    tx = box_ref[0, 1]
    th = box_ref[0, 2]
    tw = box_ref[0, 3]
    a0 = anc_ref[0, 0]
    a1 = anc_ref[0, 1]
    a2 = anc_ref[0, 2]
    a3 = anc_ref[0, 3]

    ycenter_a = (a0 + a2) * 0.5
    xcenter_a = (a1 + a3) * 0.5
    ha = a2 - a0
    wa = a3 - a1
    w = jnp.exp(jnp.clip(tw, -10.0, 4.0)) * wa
    h = jnp.exp(jnp.clip(th, -10.0, 4.0)) * ha
    ycenter = ty * ha + ycenter_a
    xcenter = tx * wa + xcenter_a
    y0 = ycenter - h * 0.5
    x0 = xcenter - w * 0.5
    y1 = ycenter + h * 0.5
    x1 = xcenter + w * 0.5

    offs = cls_f * 4096.0
    y0n = y0 + offs
    x0n = x0 + offs
    y1n = y1 + offs
    x1n = x1 + offs
    areas = (y1n - y0n) * (x1n - x0n)

    scores0 = jax.nn.sigmoid(logits)

    flat_iota = (jax.lax.broadcasted_iota(jnp.int32, (40, 128), 0) * 128
                 + jax.lax.broadcasted_iota(jnp.int32, (40, 128), 1))
    r_io = jax.lax.broadcasted_iota(jnp.int32, (128, 128), 0)
    l_io = jax.lax.broadcasted_iota(jnp.int32, (128, 128), 1)

    def step(t, carry):
        s, dets = carry
        m = jnp.max(s)
        i = jnp.min(jnp.where(s == m, flat_iota, jnp.int32(2 ** 30)))
        oh = flat_iota == i
        sel_y0 = jnp.sum(jnp.where(oh, y0, 0.0))
        sel_x0 = jnp.sum(jnp.where(oh, x0, 0.0))
        sel_y1 = jnp.sum(jnp.where(oh, y1, 0.0))
        sel_x1 = jnp.sum(jnp.where(oh, x1, 0.0))
        sel_c = jnp.sum(jnp.where(oh, cls_f, 0.0))

        off_r = sel_c * 4096.0
        r0 = sel_y0 + off_r
        r1 = sel_x0 + off_r
        r2 = sel_y1 + off_r
        r3 = sel_x1 + off_r
        yA = jnp.maximum(r0, y0n)
        xA = jnp.maximum(r1, x0n)
        yB = jnp.minimum(r2, y1n)
        xB = jnp.minimum(r3, x1n)
        inter = jnp.clip(yB - yA, 0.0) * jnp.clip(xB - xA, 0.0)
        area_r = (r2 - r0) * (r3 - r1)
        iou = inter / (area_r + areas - inter + 1e-8)
        s = jnp.where(iou > _IOU_THRESHOLD, -1.0, s)

        valid = jnp.where(m > 0.0, 1.0, 0.0)
        d0 = sel_x0 * scale * valid
        d1 = sel_y0 * scale * valid
        d2 = (sel_x1 - sel_x0) * scale * valid
        d3 = (sel_y1 - sel_y0) * scale * valid
        d4 = m * valid
        d5 = (sel_c + 1.0) * valid

        mt = r_io == t
        dets = jnp.where(mt & (l_io == 0), d0, dets)
        dets = jnp.where(mt & (l_io == 1), d1, dets)
        dets = jnp.where(mt & (l_io == 2), d2, dets)
        dets = jnp.where(mt & (l_io == 3), d3, dets)
        dets = jnp.where(mt & (l_io == 4), d4, dets)
        dets = jnp.where(mt & (l_io == 5), d5, dets)
        return s, dets

    dets0 = jnp.zeros((128, 128), dtype=jnp.float32)
    _, dets = jax.lax.fori_loop(0, _MAX_DETECTIONS_PER_IMAGE, step,
                                (scores0, dets0))
    out_ref[0] = dets


def kernel(cls_p3, cls_p4, cls_p5, cls_p6, cls_p7,
           box_p3, box_p4, box_p5, box_p6, box_p7, image_scales):
    cls_list = [cls_p3, cls_p4, cls_p5, cls_p6, cls_p7]
    box_list = [box_p3, box_p4, box_p5, box_p6, box_p7]
    B = cls_p3.shape[0]
    C = _NUM_CLASSES
    cls_all = jnp.concatenate(
        [jnp.transpose(c, (0, 2, 3, 1)).reshape(B, -1, C) for c in cls_list],
        axis=1)
    box_all = jnp.concatenate(
        [jnp.transpose(b, (0, 2, 3, 1)).reshape(B, -1, 4) for b in box_list],
        axis=1)
    flat = cls_all.reshape(B, -1)
    top_vals, topi = jax.lax.top_k(flat, _MAX_DETECTION_POINTS)
    idx = topi // C
    classes = topi % C
    box_topk = jnp.take_along_axis(box_all, idx[:, :, None], axis=1)
    anc_topk = jnp.take(_ANCHORS, idx, axis=0)

    pad = _NP - _MAX_DETECTION_POINTS
    logits_p = jnp.pad(top_vals, ((0, 0), (0, pad)),
                       constant_values=-1e9).reshape(B, 40, 128)
    cls_p = jnp.pad(classes.astype(jnp.float32),
                    ((0, 0), (0, pad))).reshape(B, 40, 128)
    box_p = jnp.pad(box_topk, ((0, 0), (0, pad), (0, 0)))
    box_p = jnp.transpose(box_p, (0, 2, 1)).reshape(B, 4, 40, 128)
    anc_p = jnp.pad(anc_topk, ((0, 0), (0, pad), (0, 0)))
    anc_p = jnp.transpose(anc_p, (0, 2, 1)).reshape(B, 4, 40, 128)
    scales_p = image_scales.reshape(B, 1, 1)

    out = pl.pallas_call(
        _nms_kernel,
        grid=(B,),
        in_specs=[
            pl.BlockSpec((1, 40, 128), lambda b: (b, 0, 0)),
            pl.BlockSpec((1, 4, 40, 128), lambda b: (b, 0, 0, 0)),
            pl.BlockSpec((1, 4, 40, 128), lambda b: (b, 0, 0, 0)),
            pl.BlockSpec((1, 40, 128), lambda b: (b, 0, 0)),
            pl.BlockSpec((1, 1, 1), lambda b: (b, 0, 0)),
        ],
        out_specs=pl.BlockSpec((1, 128, 128), lambda b: (b, 0, 0)),
        out_shape=jax.ShapeDtypeStruct((B, 128, 128), jnp.float32),
        compiler_params=pltpu.CompilerParams(
            dimension_semantics=("parallel",)),
    )(logits_p, box_p, anc_p, cls_p, scales_p)

    return out[:, :_MAX_DETECTIONS_PER_IMAGE, :6]


# speedup vs baseline: 1.0043x; 1.0043x over previous
import numpy as np
import jax
import jax.numpy as jnp
from jax.experimental import pallas as pl
from jax.experimental.pallas import tpu as pltpu

_MAX_DETECTION_POINTS = 5000
_MAX_DETECTIONS_PER_IMAGE = 100
_NUM_CLASSES = 90
_IOU_THRESHOLD = 0.5
_IMAGE_SIZE = 512
_MIN_LEVEL = 3
_MAX_LEVEL = 7
_NUM_SCALES = 3
_ANCHOR_SCALE = 4.0
_ASPECTS = [(1.0, 1.0), (1.4, 0.7), (0.7, 1.4)]

_NP = 5120


def _anchor_table():
    all_boxes = []
    for level in range(_MIN_LEVEL, _MAX_LEVEL + 1):
        stride = 2 ** level
        feat = _IMAGE_SIZE // stride
        configs = []
        for octave in range(_NUM_SCALES):
            for (ax, ay) in _ASPECTS:
                base = _ANCHOR_SCALE * stride * (2.0 ** (octave / float(_NUM_SCALES)))
                configs.append((base * ax / 2.0, base * ay / 2.0))
        xs = (np.arange(feat, dtype=np.float32) + 0.5) * stride
        ys = (np.arange(feat, dtype=np.float32) + 0.5) * stride
        yv, xv = np.meshgrid(ys, xs, indexing='ij')
        level_boxes = np.zeros((feat, feat, len(configs), 4), dtype=np.float32)
        for a, (hx, hy) in enumerate(configs):
            level_boxes[:, :, a, 0] = yv - hy
            level_boxes[:, :, a, 1] = xv - hx
            level_boxes[:, :, a, 2] = yv + hy
            level_boxes[:, :, a, 3] = xv + hx
        all_boxes.append(level_boxes.reshape(-1, 4))
    return np.concatenate(all_boxes, axis=0)


_ANCHORS = jnp.asarray(_anchor_table())


def _nms_kernel(s_ref, box_ref, anc_ref, cl_ref, sc_ref, out_ref):
    logits = s_ref[:, :]
    cls_f = cl_ref[:, :]
    scale = sc_ref[:, :]

    ty = box_ref[:, 0, :]
    tx = box_ref[:, 1, :]
    th = box_ref[:, 2, :]
    tw = box_ref[:, 3, :]
    a0 = anc_ref[:, 0, :]
    a1 = anc_ref[:, 1, :]
    a2 = anc_ref[:, 2, :]
    a3 = anc_ref[:, 3, :]

    ycenter_a = (a0 + a2) * 0.5
    xcenter_a = (a1 + a3) * 0.5
    ha = a2 - a0
    wa = a3 - a1
    w = jnp.exp(jnp.clip(tw, -10.0, 4.0)) * wa
    h = jnp.exp(jnp.clip(th, -10.0, 4.0)) * ha
    ycenter = ty * ha + ycenter_a
    xcenter = tx * wa + xcenter_a
    y0 = ycenter - h * 0.5
    x0 = xcenter - w * 0.5
    y1 = ycenter + h * 0.5
    x1 = xcenter + w * 0.5

    offs = cls_f * 4096.0
    y0n = y0 + offs
    x0n = x0 + offs
    y1n = y1 + offs
    x1n = x1 + offs
    areas = (y1n - y0n) * (x1n - x0n)

    scores0 = jax.nn.sigmoid(logits)

    B = scores0.shape[0]
    cand_iota = jax.lax.broadcasted_iota(jnp.int32, (B, _NP), 1)
    det_lane = jax.lax.broadcasted_iota(jnp.int32, (B, 128), 1)

    def step(t, carry):
        s, d0a, d1a, d2a, d3a, d4a, d5a = carry
        m = jnp.max(s, axis=1, keepdims=True)
        i = jnp.min(jnp.where(s == m, cand_iota, jnp.int32(2 ** 30)),
                    axis=1, keepdims=True)
        oh = cand_iota == i
        sel_y0 = jnp.sum(jnp.where(oh, y0, 0.0), axis=1, keepdims=True)
        sel_x0 = jnp.sum(jnp.where(oh, x0, 0.0), axis=1, keepdims=True)
        sel_y1 = jnp.sum(jnp.where(oh, y1, 0.0), axis=1, keepdims=True)
        sel_x1 = jnp.sum(jnp.where(oh, x1, 0.0), axis=1, keepdims=True)
        sel_c = jnp.sum(jnp.where(oh, cls_f, 0.0), axis=1, keepdims=True)

        off_r = sel_c * 4096.0
        r0 = sel_y0 + off_r
        r1 = sel_x0 + off_r
        r2 = sel_y1 + off_r
        r3 = sel_x1 + off_r
        yA = jnp.maximum(r0, y0n)
        xA = jnp.maximum(r1, x0n)
        yB = jnp.minimum(r2, y1n)
        xB = jnp.minimum(r3, x1n)
        inter = jnp.maximum(yB - yA, 0.0) * jnp.maximum(xB - xA, 0.0)
        area_r = (r2 - r0) * (r3 - r1)
        iou = inter / (area_r + areas - inter + 1e-8)
        s = jnp.where(iou > _IOU_THRESHOLD, -1.0, s)

        valid = jnp.where(m > 0.0, 1.0, 0.0)
        lm = det_lane == t
        d0a = jnp.where(lm, sel_x0 * scale * valid, d0a)
        d1a = jnp.where(lm, sel_y0 * scale * valid, d1a)
        d2a = jnp.where(lm, (sel_x1 - sel_x0) * scale * valid, d2a)
        d3a = jnp.where(lm, (sel_y1 - sel_y0) * scale * valid, d3a)
        d4a = jnp.where(lm, m * valid, d4a)
        d5a = jnp.where(lm, (sel_c + 1.0) * valid, d5a)
        return s, d0a, d1a, d2a, d3a, d4a, d5a

    z = jnp.zeros((B, 128), dtype=jnp.float32)
    carry = jax.lax.fori_loop(0, _MAX_DETECTIONS_PER_IMAGE, step,
                              (scores0, z, z, z, z, z, z))
    for k in range(6):
        out_ref[:, k, :] = carry[1 + k]
    out_ref[:, 6, :] = z
    out_ref[:, 7, :] = z


def kernel(cls_p3, cls_p4, cls_p5, cls_p6, cls_p7,
           box_p3, box_p4, box_p5, box_p6, box_p7, image_scales):
    cls_list = [cls_p3, cls_p4, cls_p5, cls_p6, cls_p7]
    box_list = [box_p3, box_p4, box_p5, box_p6, box_p7]
    B = cls_p3.shape[0]
    C = _NUM_CLASSES
    cls_all = jnp.concatenate(
        [jnp.transpose(c, (0, 2, 3, 1)).reshape(B, -1, C) for c in cls_list],
        axis=1)
    box_all = jnp.concatenate(
        [jnp.transpose(b, (0, 2, 3, 1)).reshape(B, -1, 4) for b in box_list],
        axis=1)
    flat = cls_all.reshape(B, -1)
    top_vals, topi = jax.lax.top_k(flat, _MAX_DETECTION_POINTS)
    idx = topi // C
    classes = topi % C
    box_topk = jnp.take_along_axis(box_all, idx[:, :, None], axis=1)
    anc_topk = jnp.take(_ANCHORS, idx, axis=0)

    pad = _NP - _MAX_DETECTION_POINTS
    logits_p = jnp.pad(top_vals, ((0, 0), (0, pad)), constant_values=-1e9)
    cls_p = jnp.pad(classes.astype(jnp.float32), ((0, 0), (0, pad)))
    box_p = jnp.transpose(jnp.pad(box_topk, ((0, 0), (0, pad), (0, 0))),
                          (0, 2, 1))
    anc_p = jnp.transpose(jnp.pad(anc_topk, ((0, 0), (0, pad), (0, 0))),
                          (0, 2, 1))
    scales_p = image_scales.reshape(B, 1)

    out = pl.pallas_call(
        _nms_kernel,
        out_shape=jax.ShapeDtypeStruct((B, 8, 128), jnp.float32),
    )(logits_p, box_p, anc_p, cls_p, scales_p)

    return jnp.transpose(out[:, :6, :_MAX_DETECTIONS_PER_IMAGE], (0, 2, 1))
